# Initial kernel scaffold; baseline (speedup 1.0000x reference)
import jax
import jax.numpy as jnp
from jax.experimental import pallas as pl


def kernel(hidden_states, prompt_lens):
    raise NotImplementedError("write your pallas kernel here")


# speedup vs baseline: 10.3975x; 10.3975x over previous
import functools

import jax
import jax.numpy as jnp
from jax import lax
from jax.experimental import pallas as pl
from jax.experimental.pallas import tpu as pltpu
from jax.experimental.pallas import tpu_sc as plsc

TOKENS = 32768
D = 1024
NSEG = 16
R = 64
C = 8
STEP = R * C
NB = TOKENS // R
NSTEPS = TOKENS // STEP
NW = 2 * NSEG
LANES = 16


def _block_sums_body(seg_ref, x_ref, acc_ref):
    k = pl.program_id(0)

    @pl.when(k == 0)
    def _():
        acc_ref[...] = jnp.zeros_like(acc_ref)

    x = x_ref[...]
    subs = jnp.sum(x.reshape(C, R, D), axis=1)
    iota = lax.broadcasted_iota(jnp.int32, (NSEG, 1), 0)
    total = jnp.zeros((NSEG, D), jnp.float32)
    for i in range(C):
        s = seg_ref[k * C + i]
        total = total + jnp.where(iota == s, subs[i:i + 1, :], 0.0)
    acc_ref[...] += total


_block_sums = pl.pallas_call(
    _block_sums_body,
    grid_spec=pltpu.PrefetchScalarGridSpec(
        num_scalar_prefetch=1,
        grid=(NSTEPS,),
        in_specs=[pl.BlockSpec((STEP, D), lambda k, seg: (k, 0))],
        out_specs=pl.BlockSpec((NSEG, D), lambda k, seg: (0, 0)),
    ),
    out_shape=jax.ShapeDtypeStruct((NSEG, D), jnp.float32),
)


@functools.cache
def _make_edge_sums():
    @functools.partial(
        pl.kernel,
        out_type=jax.ShapeDtypeStruct((NW, D), jnp.float32),
        mesh=plsc.VectorSubcoreMesh(core_axis_name="c", subcore_axis_name="s"),
        scratch_types=[
            pltpu.VMEM((LANES,), jnp.int32),
            pltpu.VMEM((R, D), jnp.float32),
            pltpu.VMEM((D,), jnp.float32),
        ],
        compiler_params=pltpu.CompilerParams(needs_layout_passes=False),
    )
    def _edge_sums(x_hbm, meta_hbm, out_hbm, meta_v, buf_v, acc_v):
        wid = lax.axis_index("c") * 16 + lax.axis_index("s")
        pltpu.sync_copy(meta_hbm.at[wid], meta_v)
        mv = meta_v[...]
        iota = lax.iota(jnp.int32, LANES)

        def _scalar(pos):
            return jnp.sum(jnp.where(iota == pos, mv, 0))

        bk = _scalar(0)
        lo = _scalar(1)
        n = _scalar(2)

        @pl.loop(0, D, step=LANES)
        def _(c):
            acc_v[pl.ds(c, LANES)] = jnp.zeros((LANES,), jnp.float32)

        @pl.when(n > 0)
        def _():
            pltpu.sync_copy(x_hbm.at[pl.ds(bk * R, R)], buf_v)
            hi = lo + n
            for c in range(0, D, LANES):
                def body(i, carry, c=c):
                    return carry + buf_v[i, pl.ds(c, LANES)]
                acc_v[pl.ds(c, LANES)] = lax.fori_loop(
                    lo, hi, body, jnp.zeros((LANES,), jnp.float32))

        pltpu.sync_copy(acc_v, out_hbm.at[wid])

    return _edge_sums


def _combine_body(acc_ref, edge_ref, len_ref, o_ref):
    tot = acc_ref[...] + edge_ref[0:NSEG, :] + edge_ref[NSEG:NW, :]
    mean = tot / len_ref[...]
    norm = jnp.sqrt(jnp.sum(mean * mean, axis=1, keepdims=True))
    o_ref[...] = mean / jnp.maximum(norm, 1e-12)


_combine = pl.pallas_call(
    _combine_body,
    out_shape=jax.ShapeDtypeStruct((NSEG, D), jnp.float32),
)


def kernel(hidden_states, prompt_lens):
    lens = prompt_lens.astype(jnp.int32)
    ends = jnp.cumsum(lens)
    starts = ends - lens
    fb = (starts + R - 1) // R
    lb = ends // R

    bidx = jnp.arange(NB, dtype=jnp.int32)
    owned = (bidx[None, :] >= fb[:, None]) & (bidx[None, :] < lb[:, None])
    blockseg = jnp.where(
        jnp.any(owned, axis=0),
        jnp.argmax(owned, axis=0).astype(jnp.int32),
        jnp.int32(NSEG),
    )

    h0 = starts
    hn = jnp.minimum(fb * R, ends) - starts
    t0 = jnp.maximum(lb * R, starts)
    tn = jnp.where(lb >= fb, ends - t0, 0)
    r0 = jnp.concatenate([h0, t0])
    nrow = jnp.concatenate([hn, tn]).astype(jnp.int32)
    bk = jnp.clip(r0 // R, 0, NB - 1).astype(jnp.int32)
    lo = (r0 - bk * R).astype(jnp.int32)
    meta = jnp.zeros((NW, LANES), jnp.int32)
    meta = meta.at[:, 0].set(bk).at[:, 1].set(lo).at[:, 2].set(nrow)

    acc = _block_sums(blockseg, hidden_states)
    edges = _make_edge_sums()(hidden_states, meta)
    lens_f32 = lens.astype(jnp.float32).reshape(NSEG, 1)
    return _combine(acc, edges, lens_f32)
